# Initial kernel scaffold; baseline (speedup 1.0000x reference)
import jax
import jax.numpy as jnp
from jax.experimental import pallas as pl


def kernel(x, edge_index, batch, params):
    raise NotImplementedError("write your pallas kernel here")


# speedup vs baseline: 16.8599x; 16.8599x over previous
import functools

import jax
import jax.numpy as jnp
from jax import lax
from jax.experimental import pallas as pl
from jax.experimental.pallas import tpu as pltpu
from jax.experimental.pallas import tpu_sc as plsc

N_NODES = 10000
NPAD = 10240
NB = NPAD // 128
E_ORIG = 320000
E_PAD = 327680
EROWS = E_PAD // 128
NW = 32
RPW = EROWS // NW
F = 128


def _col(row):
    r = jnp.broadcast_to(row, (128, 128))
    i0 = lax.broadcasted_iota(jnp.int32, (128, 128), 0)
    i1 = lax.broadcasted_iota(jnp.int32, (128, 128), 1)
    return jnp.sum(jnp.where(i0 == i1, r, 0.0), axis=1, keepdims=True)


def _rowof(col):
    c = jnp.broadcast_to(col, (128, 128))
    i0 = lax.broadcasted_iota(jnp.int32, (128, 128), 0)
    i1 = lax.broadcasted_iota(jnp.int32, (128, 128), 1)
    return jnp.sum(jnp.where(i0 == i1, c, 0.0), axis=0, keepdims=True)


def _mm_body(x_ref, s_ref, w_ref, as_ref, ad_ref, g_ref, oas_ref, oad_ref):
    xb = x_ref[...] * _col(s_ref[0])
    g = jnp.dot(xb, w_ref[...], preferred_element_type=jnp.float32)
    g_ref[...] = g
    oas_ref[0] = _rowof(jnp.sum(g * as_ref[...], axis=1, keepdims=True))
    oad_ref[0] = _rowof(jnp.sum(g * ad_ref[...], axis=1, keepdims=True))


def _tk_mm(x, s2d, w, att_s, att_d):
    return pl.pallas_call(
        _mm_body,
        grid=(NB,),
        in_specs=[
            pl.BlockSpec((128, F), lambda i: (i, 0)),
            pl.BlockSpec((1, 1, 128), lambda i: (i, 0, 0)),
            pl.BlockSpec((F, F), lambda i: (0, 0)),
            pl.BlockSpec((1, F), lambda i: (0, 0)),
            pl.BlockSpec((1, F), lambda i: (0, 0)),
        ],
        out_specs=[
            pl.BlockSpec((128, F), lambda i: (i, 0)),
            pl.BlockSpec((1, 1, 128), lambda i: (i, 0, 0)),
            pl.BlockSpec((1, 1, 128), lambda i: (i, 0, 0)),
        ],
        out_shape=[
            jax.ShapeDtypeStruct((NPAD, F), jnp.float32),
            jax.ShapeDtypeStruct((NB, 1, 128), jnp.float32),
            jax.ShapeDtypeStruct((NB, 1, 128), jnp.float32),
        ],
    )(x, s2d, w, att_s, att_d)


def _postdeg_body(p0_ref, p1_ref, h_ref, dinv_ref, hs_ref):
    deg = p0_ref[0] + p1_ref[0] + 1.0
    dinv = lax.rsqrt(deg)
    dinv_ref[0] = dinv
    hs_ref[...] = h_ref[...] * _col(dinv)


def _tk_postdeg(p0, p1, h):
    return pl.pallas_call(
        _postdeg_body,
        grid=(NB,),
        in_specs=[
            pl.BlockSpec((1, 1, 128), lambda i: (i, 0, 0)),
            pl.BlockSpec((1, 1, 128), lambda i: (i, 0, 0)),
            pl.BlockSpec((128, F), lambda i: (i, 0)),
        ],
        out_specs=[
            pl.BlockSpec((1, 1, 128), lambda i: (i, 0, 0)),
            pl.BlockSpec((128, F), lambda i: (i, 0)),
        ],
        out_shape=[
            jax.ShapeDtypeStruct((NB, 1, 128), jnp.float32),
            jax.ShapeDtypeStruct((NPAD, F), jnp.float32),
        ],
    )(p0, p1, h)


def _post_body(a0_ref, a1_ref, base_ref, mult_ref, selfv_ref, bias_ref,
               pw_ref, out_ref, sc_ref):
    agg = a0_ref[...] + a1_ref[...]
    out = (agg * _col(mult_ref[0]) + base_ref[...] * _col(selfv_ref[0])
           + bias_ref[...])
    out_ref[...] = out
    w = pw_ref[...]
    wn = jnp.sqrt(jnp.sum(w * w)) + 1e-16
    sarg = jnp.sum(out * w, axis=1, keepdims=True) / wn
    sc_ref[0] = _rowof(jnp.tanh(sarg))


def _tk_post(a0, a1, base, mult, selfv, bias, poolw):
    return pl.pallas_call(
        _post_body,
        grid=(NB,),
        in_specs=[
            pl.BlockSpec((128, F), lambda i: (i, 0)),
            pl.BlockSpec((128, F), lambda i: (i, 0)),
            pl.BlockSpec((128, F), lambda i: (i, 0)),
            pl.BlockSpec((1, 1, 128), lambda i: (i, 0, 0)),
            pl.BlockSpec((1, 1, 128), lambda i: (i, 0, 0)),
            pl.BlockSpec((1, F), lambda i: (0, 0)),
            pl.BlockSpec((1, F), lambda i: (0, 0)),
        ],
        out_specs=[
            pl.BlockSpec((128, F), lambda i: (i, 0)),
            pl.BlockSpec((1, 1, 128), lambda i: (i, 0, 0)),
        ],
        out_shape=[
            jax.ShapeDtypeStruct((NPAD, F), jnp.float32),
            jax.ShapeDtypeStruct((NB, 1, 128), jnp.float32),
        ],
    )(a0, a1, base, mult, selfv, bias, poolw)


def _sel_body(k, sc_ref, al_ref, sel_ref, scale_ref):
    score = sc_ref[...]
    alive = al_ref[...]
    bits = lax.bitcast_convert_type(score, jnp.uint32)
    m = jnp.where(bits >> 31 != jnp.uint32(0),
                  jnp.uint32(0xFFFFFFFF), jnp.uint32(0x80000000))
    key = jnp.where(alive > 0.0, bits ^ m, jnp.uint32(0))

    def tstep(i, t):
        b = jnp.uint32(31) - jnp.uint32(i)
        cand = t | (jnp.uint32(1) << b)
        cnt = jnp.sum(jnp.where(key >= cand, 1, 0))
        return jnp.where(cnt >= k, cand, t)

    T = lax.fori_loop(0, 32, tstep, jnp.uint32(0))
    cnt_gt = jnp.sum(jnp.where(key > T, 1, 0))
    r = k - cnt_gt
    i0 = lax.broadcasted_iota(jnp.int32, (NB, 128), 0)
    i1 = lax.broadcasted_iota(jnp.int32, (NB, 128), 1)
    idx = i0 * 128 + i1
    eq = key == T

    def istep(i, t):
        b = jnp.int32(13) - jnp.int32(i)
        cand = t | (jnp.int32(1) << b)
        f = jnp.sum(jnp.where(eq & (idx < cand), 1, 0))
        return jnp.where(f <= r, cand, t)

    I = lax.fori_loop(0, 14, istep, jnp.int32(0))
    sel = jnp.where((key > T) | (eq & (idx < I)), 1.0, 0.0)
    sel_ref[...] = sel
    scale_ref[...] = sel * score


def _tk_sel(score2d, alive2d, k):
    return pl.pallas_call(
        functools.partial(_sel_body, k),
        out_shape=[
            jax.ShapeDtypeStruct((NB, 128), jnp.float32),
            jax.ShapeDtypeStruct((NB, 128), jnp.float32),
        ],
    )(score2d, alive2d)


def _stab_body(as_ref, ad_ref, exps_ref, mv_ref):
    a_s = as_ref[...]
    a_d = ad_ref[...]
    t = jnp.max(a_s) + jnp.max(a_d)
    M = jnp.where(t > 0.0, t, 0.2 * t)
    es = a_s + a_d
    es = jnp.where(es > 0.0, es, 0.2 * es)
    exps_ref[...] = jnp.exp(es - M)
    mv_ref[...] = jnp.full((1, 128), M, jnp.float32)


def _tk_stab(as2d, ad2d):
    return pl.pallas_call(
        _stab_body,
        out_shape=[
            jax.ShapeDtypeStruct((NB, 128), jnp.float32),
            jax.ShapeDtypeStruct((1, 128), jnp.float32),
        ],
    )(as2d, ad2d)


def _postden_body(p0_ref, p1_ref, exps_ref, rden_ref, selfw_ref):
    denom = p0_ref[...] + p1_ref[...] + exps_ref[...]
    rden = 1.0 / denom
    rden_ref[...] = rden
    selfw_ref[...] = exps_ref[...] * rden


def _tk_postden(p0, p1, exps):
    return pl.pallas_call(
        _postden_body,
        out_shape=[
            jax.ShapeDtypeStruct((NB, 128), jnp.float32),
            jax.ShapeDtypeStruct((NB, 128), jnp.float32),
        ],
    )(p0, p1, exps)


def _final_body(h_ref, scale_ref, muw_ref, mub_ref, lvw_ref, lvb_ref,
                ldw_ref, ldb_ref, d2w_ref, d2b_ref, d1w_ref, d1b_ref,
                d0w_ref, d0b_ref, eps_ref, zz_ref, mu_ref, lv_ref):
    def step(i, acc):
        blk = h_ref[pl.ds(i * 128, 128), :]
        srow = scale_ref[i]
        return acc + jnp.sum(blk * _col(srow), axis=0, keepdims=True)

    z = lax.fori_loop(0, NB, step, jnp.zeros((1, F), jnp.float32)) / 1250.0
    zb = jnp.broadcast_to(z, (8, F))
    mu = jnp.dot(zb, muw_ref[...], preferred_element_type=jnp.float32) + mub_ref[...]
    lv = jnp.dot(zb, lvw_ref[...], preferred_element_type=jnp.float32) + lvb_ref[...]
    std = jnp.exp(0.5 * lv)
    zz = mu + eps_ref[...] * std
    zz = jnp.dot(zz, ldw_ref[...], preferred_element_type=jnp.float32) + ldb_ref[...]
    zz = jnp.dot(zz, d2w_ref[...], preferred_element_type=jnp.float32) + d2b_ref[...]
    zz = jnp.dot(zz, d1w_ref[...], preferred_element_type=jnp.float32) + d1b_ref[...]
    zz = jnp.dot(zz, d0w_ref[...], preferred_element_type=jnp.float32) + d0b_ref[...]
    zz_ref[...] = zz
    mu_ref[...] = mu
    lv_ref[...] = lv


def _tk_final(h, scale3d, p, eps8):
    return pl.pallas_call(
        _final_body,
        out_shape=[
            jax.ShapeDtypeStruct((8, F), jnp.float32),
            jax.ShapeDtypeStruct((8, 32), jnp.float32),
            jax.ShapeDtypeStruct((8, 32), jnp.float32),
        ],
    )(h, scale3d, p['mu_W'], p['mu_b'].reshape(1, 32),
      p['lv_W'], p['lv_b'].reshape(1, 32),
      p['ld_W'], p['ld_b'].reshape(1, F),
      p['dec2_W'], p['dec2_b'].reshape(1, F),
      p['dec1_W'], p['dec1_b'].reshape(1, F),
      p['dec0_W'], p['dec0_b'].reshape(1, F), eps8)


def _sca_body(src_hbm, dst_hbm, as_hbm, ad_hbm, al_hbm, m_hbm, out_hbm,
              as_v, ad_v, al_v, m_v, sidx, didx, pbuf, zbuf, acc_sh, sem):
    c = lax.axis_index("c")
    s = lax.axis_index("s")
    w = s * 2 + c
    pltpu.sync_copy(as_hbm, as_v)
    pltpu.sync_copy(ad_hbm, ad_v)
    pltpu.sync_copy(al_hbm, al_v)
    pltpu.sync_copy(m_hbm, m_v)

    def zstep(j, _):
        zbuf[pl.ds(j * 16, 16)] = jnp.zeros((16,), jnp.float32)
        return 0

    lax.fori_loop(0, 40, zstep, 0)
    pltpu.sync_copy(zbuf, acc_sh.at[pl.ds(s * 640, 640)])
    plsc.subcore_barrier()
    m = m_v[...][0]

    def row_step(t, _):
        row = w * RPW + t
        pltpu.sync_copy(src_hbm.at[row], sidx)
        pltpu.sync_copy(dst_hbm.at[row], didx)

        def grp(j, _2):
            si = sidx[pl.ds(j * 16, 16)]
            di = didx[pl.ds(j * 16, 16)]
            asg = plsc.load_gather(as_v, [si])
            adg = plsc.load_gather(ad_v, [di])
            als = plsc.load_gather(al_v, [si])
            ald = plsc.load_gather(al_v, [di])
            e = asg + adg
            e = jnp.where(e > 0.0, e, 0.2 * e)
            pbuf[pl.ds(j * 16, 16)] = jnp.exp(e - m) * als * ald
            return 0

        lax.fori_loop(0, 8, grp, 0)
        pltpu.async_copy(pbuf, acc_sh.at[didx], sem, add=True).wait()
        return 0

    lax.fori_loop(0, RPW, row_step, 0)
    plsc.subcore_barrier()
    pltpu.sync_copy(acc_sh.at[pl.ds(s * 640, 640)],
                    out_hbm.at[c, pl.ds(s * 640, 640)])


def _sc_scalar_pass(src2d, dst2d, as1d, ad1d, al1d, mv):
    mesh = plsc.VectorSubcoreMesh(core_axis_name="c", subcore_axis_name="s")
    kern = pl.kernel(
        _sca_body, mesh=mesh,
        compiler_params=pltpu.CompilerParams(needs_layout_passes=False, use_tc_tiling_on_sc=False),
        out_type=jax.ShapeDtypeStruct((2, NPAD), jnp.float32),
        scratch_types=[
            pltpu.VMEM((NPAD,), jnp.float32),
            pltpu.VMEM((NPAD,), jnp.float32),
            pltpu.VMEM((NPAD,), jnp.float32),
            pltpu.VMEM((16,), jnp.float32),
            pltpu.VMEM((128,), jnp.int32),
            pltpu.VMEM((128,), jnp.int32),
            pltpu.VMEM((128,), jnp.float32),
            pltpu.VMEM((640,), jnp.float32),
            pltpu.VMEM_SHARED((NPAD,), jnp.float32),
            pltpu.SemaphoreType.DMA,
        ],
    )
    return kern(src2d, dst2d, as1d, ad1d, al1d, mv)


HF = 64


def _make_scb_body(scaled):
    def body(src_hbm, dst_hbm, ga_hbm, gb_hbm, as_hbm, ad_hbm, al_hbm,
             rd_hbm, m_hbm,
             out_hbm, as_v, ad_v, al_v, rd_v, m_v,
             si0, si1, di0, di1, rows0, rows1, zb, acc_sh,
             g0, g1, s0, s1):
        c = lax.axis_index("c")
        s = lax.axis_index("s")
        w = s * 2 + c
        if scaled:
            pltpu.sync_copy(as_hbm, as_v)
            pltpu.sync_copy(ad_hbm, ad_v)
            pltpu.sync_copy(al_hbm, al_v)
            pltpu.sync_copy(rd_hbm, rd_v)
            pltpu.sync_copy(m_hbm, m_v)

        def zrow(i, _):
            for kk in range(HF // 16):
                zb[i, pl.ds(kk * 16, 16)] = jnp.zeros((16,), jnp.float32)
            return 0

        lax.fori_loop(0, 128, zrow, 0)

        def zcp(r, _):
            pltpu.sync_copy(zb, acc_sh.at[pl.ds(s * 640 + r * 128, 128)])
            return 0

        lax.fori_loop(0, 5, zcp, 0)
        plsc.subcore_barrier()
        m = m_v[...][0] if scaled else 0.0
        sems_g = (g0, g1)
        sems_s = (s0, s1)
        sis = (si0, si1)
        dis = (di0, di1)
        rows = (rows0, rows1)

        for ph, gh_hbm in ((0, ga_hbm), (1, gb_hbm)):
            def issue_gather(t, p):
                row = w * RPW + t
                pltpu.sync_copy(src_hbm.at[row], sis[p])
                pltpu.sync_copy(dst_hbm.at[row], dis[p])
                pltpu.async_copy(gh_hbm.at[sis[p]], rows[p], sems_g[p])

            issue_gather(0, 0)

            def outer(o, _):
                for b in range(2):
                    t = o * 2 + b
                    pltpu.make_async_copy(gh_hbm.at[sis[b]], rows[b],
                                          sems_g[b]).wait()

                    @pl.when(t + 1 < RPW)
                    def _pref():
                        @pl.when(t >= 1)
                        def _wscat():
                            pltpu.make_async_copy(
                                rows[1 - b], acc_sh.at[dis[1 - b]],
                                sems_s[1 - b]).wait()

                        issue_gather(t + 1, 1 - b)

                    if scaled:
                        def grp(j, _2):
                            si = sis[b][pl.ds(j * 16, 16)]
                            di = dis[b][pl.ds(j * 16, 16)]
                            asg = plsc.load_gather(as_v, [si])
                            adg = plsc.load_gather(ad_v, [di])
                            als = plsc.load_gather(al_v, [si])
                            ald = plsc.load_gather(al_v, [di])
                            rdd = plsc.load_gather(rd_v, [di])
                            e = asg + adg
                            e = jnp.where(e > 0.0, e, 0.2 * e)
                            cf = jnp.exp(e - m) * als * ald * rdd
                            for e16 in range(16):
                                cs = jnp.full((16,), cf[e16], jnp.float32)
                                ri = j * 16 + e16
                                for kk in range(HF // 16):
                                    sl = pl.ds(kk * 16, 16)
                                    rows[b][ri, sl] = rows[b][ri, sl] * cs
                            return 0

                        lax.fori_loop(0, 8, grp, 0)
                    pltpu.async_copy(rows[b], acc_sh.at[dis[b]], sems_s[b],
                                     add=True)
                return 0

            lax.fori_loop(0, RPW // 2, outer, 0)
            pltpu.make_async_copy(rows[0], acc_sh.at[dis[0]],
                                  sems_s[0]).wait()
            pltpu.make_async_copy(rows[1], acc_sh.at[dis[1]],
                                  sems_s[1]).wait()
            plsc.subcore_barrier()

            def dumpz(r, _):
                sl = pl.ds(s * 640 + r * 128, 128)
                pltpu.sync_copy(acc_sh.at[sl], out_hbm.at[c, ph, sl])
                pltpu.sync_copy(zb, acc_sh.at[sl])
                return 0

            lax.fori_loop(0, 5, dumpz, 0)
            plsc.subcore_barrier()

    return body


def _sc_vector_pass(src2d, dst2d, ga, gb, as1d, ad1d, al1d, rd1d, mv, scaled):
    mesh = plsc.VectorSubcoreMesh(core_axis_name="c", subcore_axis_name="s")
    kern = pl.kernel(
        _make_scb_body(scaled), mesh=mesh,
        compiler_params=pltpu.CompilerParams(needs_layout_passes=False, use_tc_tiling_on_sc=False),
        out_type=jax.ShapeDtypeStruct((2, 2, NPAD, HF), jnp.float32),
        scratch_types=[
            pltpu.VMEM((NPAD,), jnp.float32),
            pltpu.VMEM((NPAD,), jnp.float32),
            pltpu.VMEM((NPAD,), jnp.float32),
            pltpu.VMEM((NPAD,), jnp.float32),
            pltpu.VMEM((16,), jnp.float32),
            pltpu.VMEM((128,), jnp.int32),
            pltpu.VMEM((128,), jnp.int32),
            pltpu.VMEM((128,), jnp.int32),
            pltpu.VMEM((128,), jnp.int32),
            pltpu.VMEM((128, HF), jnp.float32),
            pltpu.VMEM((128, HF), jnp.float32),
            pltpu.VMEM((128, HF), jnp.float32),
            pltpu.VMEM_SHARED((NPAD, HF), jnp.float32),
            pltpu.SemaphoreType.DMA,
            pltpu.SemaphoreType.DMA,
            pltpu.SemaphoreType.DMA,
            pltpu.SemaphoreType.DMA,
        ],
    )
    return kern(src2d, dst2d, ga, gb, as1d, ad1d, al1d, rd1d, mv)


def kernel(x, edge_index, batch, params):
    f32 = jnp.float32
    xp = jnp.pad(x, ((0, NPAD - N_NODES), (0, 0)))
    npadedge = E_PAD - E_ORIG
    padid = N_NODES + (jnp.arange(npadedge, dtype=jnp.int32) % 240)
    src = jnp.concatenate([edge_index[0], padid]).reshape(EROWS, 128)
    dst = jnp.concatenate([edge_index[1], padid]).reshape(EROWS, 128)

    ones1 = jnp.ones((NPAD,), f32)
    zeros1 = jnp.zeros((NPAD,), f32)
    zero_m = jnp.zeros((16,), f32)
    alive = jnp.pad(jnp.ones((N_NODES,), f32), (0, NPAD - N_NODES))
    zrow = jnp.zeros((1, F), f32)

    def r3(a2d):
        return a2d.reshape(NB, 1, 128)

    def flat(a2d):
        return a2d.reshape(NPAD)

    h0, _, _ = _tk_mm(xp, r3(jnp.ones((NB, 128), f32)),
                      params['enc0_W'], zrow, zrow)
    degp = _sc_scalar_pass(src, dst, zeros1, zeros1, ones1, zero_m)
    dinv3, hs = _tk_postdeg(degp[0].reshape(NB, 1, 128),
                            degp[1].reshape(NB, 1, 128), h0)
    aggp = _sc_vector_pass(src, dst, hs[:, :HF], hs[:, HF:], zeros1, zeros1,
                           ones1, ones1, zero_m, scaled=False)
    aggp = jnp.concatenate([aggp[:, 0], aggp[:, 1]], axis=-1)
    dinv2d = dinv3.reshape(NB, 128)
    out, score3 = _tk_post(aggp[0], aggp[1], h0, dinv3,
                           r3(dinv2d * dinv2d),
                           params['enc0_b'].reshape(1, F),
                           params['pool0_w'].reshape(1, F))
    alive2d = alive.reshape(NB, 128)
    k = N_NODES
    hcur = out
    sc2d = score3.reshape(NB, 128)

    for lvl in (1, 2):
        k = (k + 1) // 2
        sel2d, scale2d = _tk_sel(sc2d, alive2d, k)
        g, as3, ad3 = _tk_mm(hcur, r3(scale2d), params['enc%d_W' % lvl],
                             params['enc%d_att_src' % lvl].reshape(1, F),
                             params['enc%d_att_dst' % lvl].reshape(1, F))
        as2d = as3.reshape(NB, 128)
        ad2d = ad3.reshape(NB, 128)
        exps2d, mv128 = _tk_stab(as2d, ad2d)
        mv = mv128[0, :16]
        al1 = flat(sel2d)
        denp = _sc_scalar_pass(src, dst, flat(as2d), flat(ad2d), al1, mv)
        rden2d, selfw2d = _tk_postden(denp[0].reshape(NB, 128),
                                      denp[1].reshape(NB, 128), exps2d)
        aggp = _sc_vector_pass(src, dst, g[:, :HF], g[:, HF:], flat(as2d),
                               flat(ad2d), al1, flat(rden2d), mv, scaled=True)
        aggp = jnp.concatenate([aggp[:, 0], aggp[:, 1]], axis=-1)
        hcur, score3 = _tk_post(aggp[0], aggp[1], g, r3(jnp.ones((NB, 128), f32)),
                                r3(selfw2d),
                                params['enc%d_b' % lvl].reshape(1, F),
                                params['pool%d_w' % lvl].reshape(1, F))
        sc2d = score3.reshape(NB, 128)
        alive2d = sel2d

    k = (k + 1) // 2
    sel2d, scale2d = _tk_sel(sc2d, alive2d, k)
    eps = jax.random.normal(jax.random.key(42), (1, 32), dtype=f32)
    eps8 = jnp.broadcast_to(eps, (8, 32))
    zz8, mu8, lv8 = _tk_final(hcur, scale2d.reshape(NB, 1, 128), params, eps8)
    return zz8[0:1], mu8[0:1], lv8[0:1]


# speedup vs baseline: 27.3235x; 1.6206x over previous
import functools

import jax
import jax.numpy as jnp
from jax import lax
from jax.experimental import pallas as pl
from jax.experimental.pallas import tpu as pltpu
from jax.experimental.pallas import tpu_sc as plsc

N_NODES = 10000
NPAD = 10240
NB = NPAD // 128
E_ORIG = 320000
E_PAD = 327680
EROWS = E_PAD // 128
NW = 32
RPW = EROWS // NW
F = 128


def _col(row):
    r = jnp.broadcast_to(row, (128, 128))
    i0 = lax.broadcasted_iota(jnp.int32, (128, 128), 0)
    i1 = lax.broadcasted_iota(jnp.int32, (128, 128), 1)
    return jnp.sum(jnp.where(i0 == i1, r, 0.0), axis=1, keepdims=True)


def _rowof(col):
    c = jnp.broadcast_to(col, (128, 128))
    i0 = lax.broadcasted_iota(jnp.int32, (128, 128), 0)
    i1 = lax.broadcasted_iota(jnp.int32, (128, 128), 1)
    return jnp.sum(jnp.where(i0 == i1, c, 0.0), axis=0, keepdims=True)


def _mm_body(x_ref, s_ref, w_ref, as_ref, ad_ref, g_ref, oas_ref, oad_ref):
    xb = x_ref[...] * _col(s_ref[0])
    g = jnp.dot(xb, w_ref[...], preferred_element_type=jnp.float32)
    g_ref[...] = g
    oas_ref[0] = _rowof(jnp.sum(g * as_ref[...], axis=1, keepdims=True))
    oad_ref[0] = _rowof(jnp.sum(g * ad_ref[...], axis=1, keepdims=True))


def _tk_mm(x, s2d, w, att_s, att_d):
    return pl.pallas_call(
        _mm_body,
        grid=(NB,),
        in_specs=[
            pl.BlockSpec((128, F), lambda i: (i, 0)),
            pl.BlockSpec((1, 1, 128), lambda i: (i, 0, 0)),
            pl.BlockSpec((F, F), lambda i: (0, 0)),
            pl.BlockSpec((1, F), lambda i: (0, 0)),
            pl.BlockSpec((1, F), lambda i: (0, 0)),
        ],
        out_specs=[
            pl.BlockSpec((128, F), lambda i: (i, 0)),
            pl.BlockSpec((1, 1, 128), lambda i: (i, 0, 0)),
            pl.BlockSpec((1, 1, 128), lambda i: (i, 0, 0)),
        ],
        out_shape=[
            jax.ShapeDtypeStruct((NPAD, F), jnp.float32),
            jax.ShapeDtypeStruct((NB, 1, 128), jnp.float32),
            jax.ShapeDtypeStruct((NB, 1, 128), jnp.float32),
        ],
    )(x, s2d, w, att_s, att_d)


def _postdeg_body(p0_ref, p1_ref, h_ref, dinv_ref, hs_ref):
    deg = p0_ref[0] + p1_ref[0] + 1.0
    dinv = lax.rsqrt(deg)
    dinv_ref[0] = dinv
    hs_ref[...] = h_ref[...] * _col(dinv)


def _tk_postdeg(p0, p1, h):
    return pl.pallas_call(
        _postdeg_body,
        grid=(NB,),
        in_specs=[
            pl.BlockSpec((1, 1, 128), lambda i: (i, 0, 0)),
            pl.BlockSpec((1, 1, 128), lambda i: (i, 0, 0)),
            pl.BlockSpec((128, F), lambda i: (i, 0)),
        ],
        out_specs=[
            pl.BlockSpec((1, 1, 128), lambda i: (i, 0, 0)),
            pl.BlockSpec((128, F), lambda i: (i, 0)),
        ],
        out_shape=[
            jax.ShapeDtypeStruct((NB, 1, 128), jnp.float32),
            jax.ShapeDtypeStruct((NPAD, F), jnp.float32),
        ],
    )(p0, p1, h)


def _post_body(a0_ref, a1_ref, base_ref, mult_ref, selfv_ref, bias_ref,
               pw_ref, out_ref, sc_ref):
    agg = a0_ref[...] + a1_ref[...]
    out = (agg * _col(mult_ref[0]) + base_ref[...] * _col(selfv_ref[0])
           + bias_ref[...])
    out_ref[...] = out
    w = pw_ref[...]
    wn = jnp.sqrt(jnp.sum(w * w)) + 1e-16
    sarg = jnp.sum(out * w, axis=1, keepdims=True) / wn
    sc_ref[0] = _rowof(jnp.tanh(sarg))


def _tk_post(a0, a1, base, mult, selfv, bias, poolw):
    return pl.pallas_call(
        _post_body,
        grid=(NB,),
        in_specs=[
            pl.BlockSpec((128, F), lambda i: (i, 0)),
            pl.BlockSpec((128, F), lambda i: (i, 0)),
            pl.BlockSpec((128, F), lambda i: (i, 0)),
            pl.BlockSpec((1, 1, 128), lambda i: (i, 0, 0)),
            pl.BlockSpec((1, 1, 128), lambda i: (i, 0, 0)),
            pl.BlockSpec((1, F), lambda i: (0, 0)),
            pl.BlockSpec((1, F), lambda i: (0, 0)),
        ],
        out_specs=[
            pl.BlockSpec((128, F), lambda i: (i, 0)),
            pl.BlockSpec((1, 1, 128), lambda i: (i, 0, 0)),
        ],
        out_shape=[
            jax.ShapeDtypeStruct((NPAD, F), jnp.float32),
            jax.ShapeDtypeStruct((NB, 1, 128), jnp.float32),
        ],
    )(a0, a1, base, mult, selfv, bias, poolw)


def _sel_body(k, sc_ref, al_ref, sel_ref, scale_ref):
    score = sc_ref[...]
    alive = al_ref[...]
    bits = lax.bitcast_convert_type(score, jnp.uint32)
    m = jnp.where(bits >> 31 != jnp.uint32(0),
                  jnp.uint32(0xFFFFFFFF), jnp.uint32(0x80000000))
    key = jnp.where(alive > 0.0, bits ^ m, jnp.uint32(0))

    def tstep(i, t):
        b = jnp.uint32(31) - jnp.uint32(i)
        cand = t | (jnp.uint32(1) << b)
        cnt = jnp.sum(jnp.where(key >= cand, 1, 0))
        return jnp.where(cnt >= k, cand, t)

    T = lax.fori_loop(0, 32, tstep, jnp.uint32(0))
    cnt_gt = jnp.sum(jnp.where(key > T, 1, 0))
    r = k - cnt_gt
    i0 = lax.broadcasted_iota(jnp.int32, (NB, 128), 0)
    i1 = lax.broadcasted_iota(jnp.int32, (NB, 128), 1)
    idx = i0 * 128 + i1
    eq = key == T

    def istep(i, t):
        b = jnp.int32(13) - jnp.int32(i)
        cand = t | (jnp.int32(1) << b)
        f = jnp.sum(jnp.where(eq & (idx < cand), 1, 0))
        return jnp.where(f <= r, cand, t)

    I = lax.fori_loop(0, 14, istep, jnp.int32(0))
    sel = jnp.where((key > T) | (eq & (idx < I)), 1.0, 0.0)
    sel_ref[...] = sel
    scale_ref[...] = sel * score


def _tk_sel(score2d, alive2d, k):
    return pl.pallas_call(
        functools.partial(_sel_body, k),
        out_shape=[
            jax.ShapeDtypeStruct((NB, 128), jnp.float32),
            jax.ShapeDtypeStruct((NB, 128), jnp.float32),
        ],
    )(score2d, alive2d)


def _stab_body(as_ref, ad_ref, exps_ref, mv_ref):
    a_s = as_ref[...]
    a_d = ad_ref[...]
    t = jnp.max(a_s) + jnp.max(a_d)
    M = jnp.where(t > 0.0, t, 0.2 * t)
    es = a_s + a_d
    es = jnp.where(es > 0.0, es, 0.2 * es)
    exps_ref[...] = jnp.exp(es - M)
    mv_ref[...] = jnp.full((1, 128), M, jnp.float32)


def _tk_stab(as2d, ad2d):
    return pl.pallas_call(
        _stab_body,
        out_shape=[
            jax.ShapeDtypeStruct((NB, 128), jnp.float32),
            jax.ShapeDtypeStruct((1, 128), jnp.float32),
        ],
    )(as2d, ad2d)


def _postden_body(p0_ref, p1_ref, exps_ref, rden_ref, selfw_ref):
    denom = p0_ref[...] + p1_ref[...] + exps_ref[...]
    rden = 1.0 / denom
    rden_ref[...] = rden
    selfw_ref[...] = exps_ref[...] * rden


def _tk_postden(p0, p1, exps):
    return pl.pallas_call(
        _postden_body,
        out_shape=[
            jax.ShapeDtypeStruct((NB, 128), jnp.float32),
            jax.ShapeDtypeStruct((NB, 128), jnp.float32),
        ],
    )(p0, p1, exps)


def _final_body(h_ref, scale_ref, muw_ref, mub_ref, lvw_ref, lvb_ref,
                ldw_ref, ldb_ref, d2w_ref, d2b_ref, d1w_ref, d1b_ref,
                d0w_ref, d0b_ref, eps_ref, zz_ref, mu_ref, lv_ref):
    def step(i, acc):
        blk = h_ref[pl.ds(i * 128, 128), :]
        srow = scale_ref[i]
        return acc + jnp.sum(blk * _col(srow), axis=0, keepdims=True)

    z = lax.fori_loop(0, NB, step, jnp.zeros((1, F), jnp.float32)) / 1250.0
    zb = jnp.broadcast_to(z, (8, F))
    mu = jnp.dot(zb, muw_ref[...], preferred_element_type=jnp.float32) + mub_ref[...]
    lv = jnp.dot(zb, lvw_ref[...], preferred_element_type=jnp.float32) + lvb_ref[...]
    std = jnp.exp(0.5 * lv)
    zz = mu + eps_ref[...] * std
    zz = jnp.dot(zz, ldw_ref[...], preferred_element_type=jnp.float32) + ldb_ref[...]
    zz = jnp.dot(zz, d2w_ref[...], preferred_element_type=jnp.float32) + d2b_ref[...]
    zz = jnp.dot(zz, d1w_ref[...], preferred_element_type=jnp.float32) + d1b_ref[...]
    zz = jnp.dot(zz, d0w_ref[...], preferred_element_type=jnp.float32) + d0b_ref[...]
    zz_ref[...] = zz
    mu_ref[...] = mu
    lv_ref[...] = lv


def _tk_final(h, scale3d, p, eps8):
    return pl.pallas_call(
        _final_body,
        out_shape=[
            jax.ShapeDtypeStruct((8, F), jnp.float32),
            jax.ShapeDtypeStruct((8, 32), jnp.float32),
            jax.ShapeDtypeStruct((8, 32), jnp.float32),
        ],
    )(h, scale3d, p['mu_W'], p['mu_b'].reshape(1, 32),
      p['lv_W'], p['lv_b'].reshape(1, 32),
      p['ld_W'], p['ld_b'].reshape(1, F),
      p['dec2_W'], p['dec2_b'].reshape(1, F),
      p['dec1_W'], p['dec1_b'].reshape(1, F),
      p['dec0_W'], p['dec0_b'].reshape(1, F), eps8)


CAP = 10496


def _sca_body(src_hbm, dst_hbm, as_hbm, ad_hbm, al_hbm, m_hbm,
              out_hbm, srcc_hbm, dstc_hbm, pc_hbm, cnt_hbm,
              as_v, ad_v, al_v, m_v, sidx, didx, pbuf, zbuf,
              sc_v, dc_v, pc_v, cntb, acc_sh, sem):
    c = lax.axis_index("c")
    s = lax.axis_index("s")
    w = s * 2 + c
    pltpu.sync_copy(as_hbm, as_v)
    pltpu.sync_copy(ad_hbm, ad_v)
    pltpu.sync_copy(al_hbm, al_v)
    pltpu.sync_copy(m_hbm, m_v)

    def zstep(j, _):
        zbuf[pl.ds(j * 16, 16)] = jnp.zeros((16,), jnp.float32)
        return 0

    lax.fori_loop(0, 40, zstep, 0)
    pltpu.sync_copy(zbuf, acc_sh.at[pl.ds(s * 640, 640)])
    plsc.subcore_barrier()
    m = m_v[...][0]

    def row_step(t, off):
        row = w * RPW + t
        pltpu.sync_copy(src_hbm.at[row], sidx)
        pltpu.sync_copy(dst_hbm.at[row], didx)

        def grp(j, off2):
            si = sidx[pl.ds(j * 16, 16)]
            di = didx[pl.ds(j * 16, 16)]
            asg = plsc.load_gather(as_v, [si])
            adg = plsc.load_gather(ad_v, [di])
            als = plsc.load_gather(al_v, [si])
            ald = plsc.load_gather(al_v, [di])
            e = asg + adg
            e = jnp.where(e > 0.0, e, 0.2 * e)
            p = jnp.exp(e - m) * als * ald
            pbuf[pl.ds(j * 16, 16)] = p
            msk = p > 0.0
            plsc.store_compressed(sc_v.at[pl.ds(off2, 16)], si, mask=msk)
            plsc.store_compressed(dc_v.at[pl.ds(off2, 16)], di, mask=msk)
            plsc.store_compressed(pc_v.at[pl.ds(off2, 16)], p, mask=msk)
            return off2 + plsc.all_reduce_population_count(msk)[0]

        off = lax.fori_loop(0, 8, grp, off)
        pltpu.async_copy(pbuf, acc_sh.at[didx], sem, add=True).wait()
        return off

    off = lax.fori_loop(0, RPW, row_step, jnp.int32(0))
    lane = jnp.arange(16, dtype=jnp.int32)
    for gi in range(16):
        sent = N_NODES + ((w * 16 + gi * 16 + lane) % 240)
        sc_v[pl.ds(off + gi * 16, 16)] = sent
        dc_v[pl.ds(off + gi * 16, 16)] = sent
        pc_v[pl.ds(off + gi * 16, 16)] = jnp.zeros((16,), jnp.float32)
    nch = 2 * (off // 256 + 1)
    cntb[pl.ds(0, 16)] = jnp.broadcast_to(nch, (16,)).astype(jnp.int32)
    pltpu.sync_copy(cntb, cnt_hbm.at[w])
    pltpu.sync_copy(sc_v, srcc_hbm.at[w])
    pltpu.sync_copy(dc_v, dstc_hbm.at[w])
    pltpu.sync_copy(pc_v, pc_hbm.at[w])
    plsc.subcore_barrier()
    pltpu.sync_copy(acc_sh.at[pl.ds(s * 640, 640)],
                    out_hbm.at[c, pl.ds(s * 640, 640)])


def _sc_scalar_pass(src2d, dst2d, as1d, ad1d, al1d, mv):
    mesh = plsc.VectorSubcoreMesh(core_axis_name="c", subcore_axis_name="s")
    kern = pl.kernel(
        _sca_body, mesh=mesh,
        compiler_params=pltpu.CompilerParams(needs_layout_passes=False, use_tc_tiling_on_sc=False),
        out_type=[
            jax.ShapeDtypeStruct((2, NPAD), jnp.float32),
            jax.ShapeDtypeStruct((NW, CAP), jnp.int32),
            jax.ShapeDtypeStruct((NW, CAP), jnp.int32),
            jax.ShapeDtypeStruct((NW, CAP), jnp.float32),
            jax.ShapeDtypeStruct((NW, 16), jnp.int32),
        ],
        scratch_types=[
            pltpu.VMEM((NPAD,), jnp.float32),
            pltpu.VMEM((NPAD,), jnp.float32),
            pltpu.VMEM((NPAD,), jnp.float32),
            pltpu.VMEM((16,), jnp.float32),
            pltpu.VMEM((128,), jnp.int32),
            pltpu.VMEM((128,), jnp.int32),
            pltpu.VMEM((128,), jnp.float32),
            pltpu.VMEM((640,), jnp.float32),
            pltpu.VMEM((CAP,), jnp.int32),
            pltpu.VMEM((CAP,), jnp.int32),
            pltpu.VMEM((CAP,), jnp.float32),
            pltpu.VMEM((16,), jnp.int32),
            pltpu.VMEM_SHARED((NPAD,), jnp.float32),
            pltpu.SemaphoreType.DMA,
        ],
    )
    return kern(src2d, dst2d, as1d, ad1d, al1d, mv)


HF = 64


def _make_scb_body(scaled):
    def body(srcc_hbm, dstc_hbm, pc_hbm, cnt_hbm, ga_hbm, gb_hbm,
             out_hbm, cntb,
             si0, si1, di0, di1, pb0, pb1, rows0, rows1, zb, acc_sh,
             g0, g1, s0, s1):
        c = lax.axis_index("c")
        s = lax.axis_index("s")
        w = s * 2 + c
        pltpu.sync_copy(cnt_hbm.at[w], cntb)
        nch = cntb[...][0]

        def zrow(i, _):
            for kk in range(HF // 16):
                zb[i, pl.ds(kk * 16, 16)] = jnp.zeros((16,), jnp.float32)
            return 0

        lax.fori_loop(0, 128, zrow, 0)

        def zcp(r, _):
            pltpu.sync_copy(zb, acc_sh.at[pl.ds(s * 640 + r * 128, 128)])
            return 0

        lax.fori_loop(0, 5, zcp, 0)
        plsc.subcore_barrier()
        sems_g = (g0, g1)
        sems_s = (s0, s1)
        sis = (si0, si1)
        dis = (di0, di1)
        pbs = (pb0, pb1)
        rows = (rows0, rows1)

        for ph, gh_hbm in ((0, ga_hbm), (1, gb_hbm)):
            def issue_gather(t, p):
                sl = pl.ds(t * 128, 128)
                pltpu.sync_copy(srcc_hbm.at[w, sl], sis[p])
                pltpu.sync_copy(dstc_hbm.at[w, sl], dis[p])
                if scaled:
                    pltpu.sync_copy(pc_hbm.at[w, sl], pbs[p])
                pltpu.async_copy(gh_hbm.at[sis[p]], rows[p], sems_g[p])

            issue_gather(0, 0)

            def outer(o, _):
                for b in range(2):
                    t = o * 2 + b
                    pltpu.make_async_copy(gh_hbm.at[sis[b]], rows[b],
                                          sems_g[b]).wait()

                    @pl.when(t + 1 < nch)
                    def _pref():
                        @pl.when(t >= 1)
                        def _wscat():
                            pltpu.make_async_copy(
                                rows[1 - b], acc_sh.at[dis[1 - b]],
                                sems_s[1 - b]).wait()

                        issue_gather(t + 1, 1 - b)

                    if scaled:
                        def grp(j, _2):
                            cf = pbs[b][pl.ds(j * 16, 16)]
                            for e16 in range(16):
                                cs = jnp.full((16,), cf[e16], jnp.float32)
                                ri = j * 16 + e16
                                for kk in range(HF // 16):
                                    sl = pl.ds(kk * 16, 16)
                                    rows[b][ri, sl] = rows[b][ri, sl] * cs
                            return 0

                        lax.fori_loop(0, 8, grp, 0)
                    pltpu.async_copy(rows[b], acc_sh.at[dis[b]], sems_s[b],
                                     add=True)
                return 0

            lax.fori_loop(0, nch // 2, outer, 0)
            pltpu.make_async_copy(rows[0], acc_sh.at[dis[0]],
                                  sems_s[0]).wait()
            pltpu.make_async_copy(rows[1], acc_sh.at[dis[1]],
                                  sems_s[1]).wait()
            plsc.subcore_barrier()

            def dumpz(r, _):
                sl = pl.ds(s * 640 + r * 128, 128)
                pltpu.sync_copy(acc_sh.at[sl], out_hbm.at[c, ph, sl])
                pltpu.sync_copy(zb, acc_sh.at[sl])
                return 0

            lax.fori_loop(0, 5, dumpz, 0)
            plsc.subcore_barrier()

    return body


def _sc_vector_pass(srcc, dstc, pcc, cnt, ga, gb, scaled):
    mesh = plsc.VectorSubcoreMesh(core_axis_name="c", subcore_axis_name="s")
    kern = pl.kernel(
        _make_scb_body(scaled), mesh=mesh,
        compiler_params=pltpu.CompilerParams(needs_layout_passes=False, use_tc_tiling_on_sc=False),
        out_type=jax.ShapeDtypeStruct((2, 2, NPAD, HF), jnp.float32),
        scratch_types=[
            pltpu.VMEM((16,), jnp.int32),
            pltpu.VMEM((128,), jnp.int32),
            pltpu.VMEM((128,), jnp.int32),
            pltpu.VMEM((128,), jnp.int32),
            pltpu.VMEM((128,), jnp.int32),
            pltpu.VMEM((128,), jnp.float32),
            pltpu.VMEM((128,), jnp.float32),
            pltpu.VMEM((128, HF), jnp.float32),
            pltpu.VMEM((128, HF), jnp.float32),
            pltpu.VMEM((128, HF), jnp.float32),
            pltpu.VMEM_SHARED((NPAD, HF), jnp.float32),
            pltpu.SemaphoreType.DMA,
            pltpu.SemaphoreType.DMA,
            pltpu.SemaphoreType.DMA,
            pltpu.SemaphoreType.DMA,
        ],
    )
    return kern(srcc, dstc, pcc, cnt, ga, gb)


def kernel(x, edge_index, batch, params):
    f32 = jnp.float32
    xp = jnp.pad(x, ((0, NPAD - N_NODES), (0, 0)))
    npadedge = E_PAD - E_ORIG
    padid = N_NODES + (jnp.arange(npadedge, dtype=jnp.int32) % 240)
    src = jnp.concatenate([edge_index[0], padid]).reshape(EROWS, 128)
    dst = jnp.concatenate([edge_index[1], padid]).reshape(EROWS, 128)

    ones1 = jnp.ones((NPAD,), f32)
    zeros1 = jnp.zeros((NPAD,), f32)
    zero_m = jnp.zeros((16,), f32)
    alive = jnp.pad(jnp.ones((N_NODES,), f32), (0, NPAD - N_NODES))
    zrow = jnp.zeros((1, F), f32)

    def r3(a2d):
        return a2d.reshape(NB, 1, 128)

    def flat(a2d):
        return a2d.reshape(NPAD)

    h0, _, _ = _tk_mm(xp, r3(jnp.ones((NB, 128), f32)),
                      params['enc0_W'], zrow, zrow)
    degp, srcc, dstc, pcc, cnt = _sc_scalar_pass(
        src, dst, zeros1, zeros1, ones1, zero_m)
    dinv3, hs = _tk_postdeg(degp[0].reshape(NB, 1, 128),
                            degp[1].reshape(NB, 1, 128), h0)
    aggp = _sc_vector_pass(srcc, dstc, pcc, cnt, hs[:, :HF], hs[:, HF:],
                           scaled=False)
    aggp = jnp.concatenate([aggp[:, 0], aggp[:, 1]], axis=-1)
    dinv2d = dinv3.reshape(NB, 128)
    out, score3 = _tk_post(aggp[0], aggp[1], h0, dinv3,
                           r3(dinv2d * dinv2d),
                           params['enc0_b'].reshape(1, F),
                           params['pool0_w'].reshape(1, F))
    alive2d = alive.reshape(NB, 128)
    k = N_NODES
    hcur = out
    sc2d = score3.reshape(NB, 128)

    for lvl in (1, 2):
        k = (k + 1) // 2
        sel2d, scale2d = _tk_sel(sc2d, alive2d, k)
        g, as3, ad3 = _tk_mm(hcur, r3(scale2d), params['enc%d_W' % lvl],
                             params['enc%d_att_src' % lvl].reshape(1, F),
                             params['enc%d_att_dst' % lvl].reshape(1, F))
        as2d = as3.reshape(NB, 128)
        ad2d = ad3.reshape(NB, 128)
        exps2d, mv128 = _tk_stab(as2d, ad2d)
        mv = mv128[0, :16]
        al1 = flat(sel2d)
        denp, srcc1, dstc1, pcc1, cnt1 = _sc_scalar_pass(
            src, dst, flat(as2d), flat(ad2d), al1, mv)
        rden2d, selfw2d = _tk_postden(denp[0].reshape(NB, 128),
                                      denp[1].reshape(NB, 128), exps2d)
        aggp = _sc_vector_pass(srcc1, dstc1, pcc1, cnt1, g[:, :HF], g[:, HF:],
                               scaled=True)
        aggp = jnp.concatenate([aggp[:, 0], aggp[:, 1]], axis=-1)
        hcur, score3 = _tk_post(aggp[0], aggp[1], g, r3(rden2d),
                                r3(selfw2d),
                                params['enc%d_b' % lvl].reshape(1, F),
                                params['pool%d_w' % lvl].reshape(1, F))
        sc2d = score3.reshape(NB, 128)
        alive2d = sel2d

    k = (k + 1) // 2
    sel2d, scale2d = _tk_sel(sc2d, alive2d, k)
    eps = jax.random.normal(jax.random.key(42), (1, 32), dtype=f32)
    eps8 = jnp.broadcast_to(eps, (8, 32))
    zz8, mu8, lv8 = _tk_final(hcur, scale2d.reshape(NB, 1, 128), params, eps8)
    return zz8[0:1], mu8[0:1], lv8[0:1]
